# baseline (speedup 1.0000x reference)
import functools

import jax
import jax.numpy as jnp
from jax import lax
from jax.experimental import pallas as pl
from jax.experimental.pallas import tpu as pltpu
from jax.experimental.pallas import tpu_sc as plsc

_NBUF = 4


def _make_gather(B, D, C):
    info = plsc.get_sparse_core_info()
    NC, NS = info.num_cores, info.num_subcores
    NW = NC * NS
    b_per_w = B // NW
    n_chunks = b_per_w // C
    n_groups = n_chunks // _NBUF
    mesh = plsc.VectorSubcoreMesh(core_axis_name="c", subcore_axis_name="s")

    @functools.partial(
        pl.kernel,
        mesh=mesh,
        out_type=jax.ShapeDtypeStruct((B, D), jnp.float32),
        scratch_types=[
            [pltpu.VMEM((C,), jnp.int32) for _ in range(_NBUF)],
            [pltpu.VMEM((C, D), jnp.float32) for _ in range(_NBUF)],
            [pltpu.SemaphoreType.DMA for _ in range(_NBUF)],
            [pltpu.SemaphoreType.DMA for _ in range(_NBUF)],
            [pltpu.SemaphoreType.DMA for _ in range(_NBUF)],
        ],
        compiler_params=pltpu.CompilerParams(use_tc_tiling_on_sc=False),
    )
    def k(idx_hbm, table_hbm, out_hbm, idx_b, rows, sem_i, sem_g, sem_w):
        wid = lax.axis_index("s") * NC + lax.axis_index("c")
        base = wid * b_per_w

        def idx_load(chunk, b):
            pltpu.async_copy(
                idx_hbm.at[pl.ds(base + chunk * C, C)], idx_b[b], sem_i[b]
            )

        def gather(b):
            pltpu.async_copy(table_hbm.at[idx_b[b]], rows[b], sem_g[b])

        def writeback(chunk, b):
            pltpu.async_copy(
                rows[b], out_hbm.at[pl.ds(base + chunk * C, C)], sem_w[b]
            )

        def wait_i(b):
            pltpu.make_async_copy(
                idx_hbm.at[pl.ds(base, C)], idx_b[b], sem_i[b]
            ).wait()

        def wait_g(b):
            pltpu.make_async_copy(
                table_hbm.at[idx_b[b]], rows[b], sem_g[b]
            ).wait()

        def wait_w(b):
            pltpu.make_async_copy(
                rows[b], out_hbm.at[pl.ds(base, C)], sem_w[b]
            ).wait()

        for b in range(_NBUF):
            idx_load(b, b)
        for b in range(_NBUF):
            wait_i(b)
            gather(b)

        def body(grp, carry):
            c0 = grp * _NBUF
            for b in range(_NBUF):
                wait_g(b)
                writeback(c0 + b, b)
                idx_load(c0 + _NBUF + b, b)
            for b in range(_NBUF):
                wait_w(b)
                wait_i(b)
                gather(b)
            return carry

        lax.fori_loop(0, n_groups - 1, body, 0)

        c0 = (n_groups - 1) * _NBUF
        for b in range(_NBUF):
            wait_g(b)
            writeback(c0 + b, b)
        for b in range(_NBUF):
            wait_w(b)

    return k


def kernel(pre_embedding, table):
    S, T = pre_embedding.shape
    V, D = table.shape
    B = S * T
    idx = pre_embedding.reshape(B).astype(jnp.int32)
    out = _make_gather(B, D, 256)(idx, table)
    return out.reshape(S, T, D)
